# baseline (speedup 1.0000x reference)
import functools

import jax
import jax.numpy as jnp
from jax import lax
from jax.experimental import pallas as pl
from jax.experimental.pallas import tpu as pltpu
from jax.experimental.pallas import tpu_sc as plsc

_NW = 32


def _gather_rows(table, idx, tc_tiling=True):
    if table.dtype == jnp.bfloat16:
        V, D = table.shape
        t32 = lax.bitcast_convert_type(
            table.reshape(V, D // 2, 2), jnp.int32)
        out32 = _gather_rows(t32, idx, tc_tiling=False)
        return lax.bitcast_convert_type(out32, jnp.bfloat16).reshape(
            idx.shape[0], D)
    B = idx.shape[0]
    V, D = table.shape
    b_per_w = B // _NW
    CH = 80
    K = 5
    n_iter = b_per_w // (K * CH)
    mesh = plsc.VectorSubcoreMesh(core_axis_name="c", subcore_axis_name="s")

    def body(table_hbm, idx_hbm, out_hbm, idx_all, rows_v, sem):
        wid = lax.axis_index("s") * 2 + lax.axis_index("c")
        base = wid * b_per_w
        pltpu.sync_copy(idx_hbm.at[pl.ds(base, b_per_w)], idx_all)

        def step(c, carry):
            off = c * (K * CH)
            hs = [pltpu.async_copy(
                      table_hbm.at[idx_all.at[pl.ds(off + j * CH, CH)]],
                      rows_v.at[pl.ds(j * CH, CH)], sem)
                  for j in range(K)]
            for h in hs:
                h.wait()
            pltpu.sync_copy(rows_v, out_hbm.at[pl.ds(base + off, K * CH)])
            return carry

        lax.fori_loop(0, n_iter, step, 0)

    gk = pl.kernel(
        body,
        out_type=jax.ShapeDtypeStruct((B, D), table.dtype),
        mesh=mesh,
        scratch_types=[
            pltpu.VMEM((b_per_w,), jnp.int32),
            pltpu.VMEM((K * CH, D), table.dtype),
            pltpu.SemaphoreType.DMA,
        ],
        compiler_params=pltpu.CompilerParams(use_tc_tiling_on_sc=tc_tiling),
    )
    return gk(table, idx)


def _pred_gather(node_last_eid, uvn, feat, ts):
    Bq = uvn.shape[0]
    N = node_last_eid.shape[0]
    E, D = feat.shape
    per = Bq // _NW
    mesh = plsc.VectorSubcoreMesh(core_axis_name="c", subcore_axis_name="s")

    def body(nle_hbm, uvn_hbm, feat_hbm, ts_hbm, rows_out, tse_out,
             uv_v, eidx_v, rows_v, tse_v, sem):
        wid = lax.axis_index("s") * 2 + lax.axis_index("c")
        base = wid * per
        pltpu.sync_copy(uvn_hbm.at[pl.ds(base, per)], uv_v)
        pltpu.async_copy(nle_hbm.at[uv_v], eidx_v, sem).wait()
        pltpu.async_copy(feat_hbm.at[eidx_v], rows_v, sem).wait()
        pltpu.async_copy(ts_hbm.at[eidx_v], tse_v, sem).wait()
        pltpu.sync_copy(rows_v, rows_out.at[pl.ds(base, per)])
        pltpu.sync_copy(tse_v, tse_out.at[pl.ds(base, per)])

    gk = pl.kernel(
        body,
        out_type=(
            jax.ShapeDtypeStruct((Bq, D), feat.dtype),
            jax.ShapeDtypeStruct((Bq,), ts.dtype),
        ),
        mesh=mesh,
        scratch_types=[
            pltpu.VMEM((per,), jnp.int32),
            pltpu.VMEM((per,), jnp.int32),
            pltpu.VMEM((per, D), feat.dtype),
            pltpu.VMEM((per,), ts.dtype),
            pltpu.SemaphoreType.DMA,
        ],
    )
    return gk(node_last_eid, uvn, feat, ts)


_BLK = 2560


def _fast_cos(z):
    nf = jnp.floor(z * 0.3183098861837907 + 0.5)
    r = (z - nf * 3.140625) - nf * 9.676535897932795e-4
    r2 = r * r
    p = 1.0 + r2 * (-0.5 + r2 * (4.1666667908e-2 + r2 * (
        -1.3888889225e-3 + r2 * (2.4801587642e-5 + r2 * -2.7557314297e-7))))
    sign = 1.0 - 2.0 * (nf.astype(jnp.int32) & 1).astype(jnp.float32)
    return sign * p


def _encode(g, ef, ts_col, Wenc1, benc1, basis1, phase1):
    E, D = g.shape
    DE = ef.shape[1]
    H = Wenc1.shape[1]
    nb = E // _BLK

    def body(g_ref, ef_ref, ts_ref, W_ref, b_ref, bas_ref, ph_ref, o_ref):
        te = _fast_cos(ts_ref[...] * bas_ref[...] + ph_ref[...])
        W = W_ref[...]
        g = g_ref[...].astype(jnp.float32)
        ef = ef_ref[...].astype(jnp.float32)
        acc = jnp.dot(g, W[0:D], preferred_element_type=jnp.float32)
        acc = acc + jnp.dot(ef, W[D:D + DE],
                            preferred_element_type=jnp.float32)
        acc = acc + jnp.dot(te, W[D + DE:D + DE + H],
                            preferred_element_type=jnp.float32)
        o_ref[...] = jnp.maximum(acc + b_ref[...], 0.0)

    return pl.pallas_call(
        body,
        grid=(nb,),
        in_specs=[
            pl.BlockSpec((_BLK, D), lambda i: (i, 0)),
            pl.BlockSpec((_BLK, DE), lambda i: (i, 0)),
            pl.BlockSpec((_BLK, 1), lambda i: (i, 0)),
            pl.BlockSpec((D + DE + H, H), lambda i: (0, 0)),
            pl.BlockSpec((1, H), lambda i: (0, 0)),
            pl.BlockSpec((1, H), lambda i: (0, 0)),
            pl.BlockSpec((1, H), lambda i: (0, 0)),
        ],
        out_specs=pl.BlockSpec((_BLK, H), lambda i: (i, 0)),
        out_shape=jax.ShapeDtypeStruct((E, H), jnp.float32),
        compiler_params=pltpu.CompilerParams(
            dimension_semantics=("arbitrary",)),
    )(g, ef, ts_col, Wenc1, benc1, basis1, phase1)


_SCH = 256


def _conv(sf, df, seg_col, seg_row, Ws, Wn, bc, out_dtype):
    E, H = df.shape
    nb = E // _BLK

    def body(sf_ref, df_ref, seg_ref, segr_ref, Ws_ref, Wn_ref, b_ref, o_ref,
             csum, ccnt, cseg):
        @pl.when(pl.program_id(0) == 0)
        def _init():
            csum[...] = jnp.zeros_like(csum)
            ccnt[...] = jnp.zeros_like(ccnt)
            cseg[...] = jnp.full_like(cseg, -1)

        seg = seg_ref[...]
        segr = segr_ref[...]
        y = sf_ref[...].astype(jnp.float32)
        ri = lax.broadcasted_iota(jnp.int32, (_SCH, _SCH), 0)
        ci = lax.broadcasted_iota(jnp.int32, (_SCH, _SCH), 1)
        tri = (ri >= ci).astype(jnp.float32)

        carry_sum = csum[...]
        carry_cnt = ccnt[...]
        carry_seg = cseg[...]
        aggs = []
        for k in range(_BLK // _SCH):
            sc = seg[k * _SCH:(k + 1) * _SCH]
            sr = segr[:, k * _SCH:(k + 1) * _SCH]
            yk = y[k * _SCH:(k + 1) * _SCH]
            M = (sc == sr).astype(jnp.float32) * tri
            yh = yk.astype(jnp.bfloat16).astype(jnp.float32)
            yl = yk - yh
            inc = (jnp.dot(M, yh, preferred_element_type=jnp.float32)
                   + jnp.dot(M, yl, preferred_element_type=jnp.float32))
            cnt = jnp.sum(M, axis=1, keepdims=True)
            cont = (sc == carry_seg).astype(jnp.float32)
            inc = inc + cont * carry_sum
            pos = cnt + cont * carry_cnt
            aggs.append(inc / pos)
            last_seg = jnp.max(sr, axis=1, keepdims=True)
            eql = (sr == last_seg).astype(jnp.float32)
            tail = (jnp.dot(eql, yh, preferred_element_type=jnp.float32)
                    + jnp.dot(eql, yl, preferred_element_type=jnp.float32))
            contl = (last_seg == carry_seg).astype(jnp.float32)
            carry_sum = tail + contl * carry_sum
            carry_cnt = (jnp.sum(eql, axis=1, keepdims=True)
                         + contl * carry_cnt)
            carry_seg = last_seg
        csum[...] = carry_sum
        ccnt[...] = carry_cnt
        cseg[...] = carry_seg

        agg = jnp.concatenate(aggs, axis=0)
        acc = jnp.dot(df_ref[...].astype(jnp.float32), Ws_ref[...],
                      preferred_element_type=jnp.float32)
        acc = acc + jnp.dot(agg, Wn_ref[...],
                            preferred_element_type=jnp.float32)
        o_ref[...] = jnp.maximum(acc + b_ref[...], 0.0).astype(out_dtype)

    return pl.pallas_call(
        body,
        grid=(nb,),
        in_specs=[
            pl.BlockSpec((_BLK, H), lambda i: (i, 0)),
            pl.BlockSpec((_BLK, H), lambda i: (i, 0)),
            pl.BlockSpec((_BLK, 1), lambda i: (i, 0)),
            pl.BlockSpec((1, _BLK), lambda i: (0, i)),
            pl.BlockSpec((H, H), lambda i: (0, 0)),
            pl.BlockSpec((H, H), lambda i: (0, 0)),
            pl.BlockSpec((1, H), lambda i: (0, 0)),
        ],
        out_specs=pl.BlockSpec((_BLK, H), lambda i: (i, 0)),
        out_shape=jax.ShapeDtypeStruct((E, H), out_dtype),
        scratch_shapes=[
            pltpu.VMEM((1, H), jnp.float32),
            pltpu.VMEM((1, 1), jnp.float32),
            pltpu.VMEM((1, 1), jnp.int32),
        ],
        compiler_params=pltpu.CompilerParams(
            dimension_semantics=("arbitrary",)),
    )(sf, df, seg_col, seg_row, Ws, Wn, bc)


def _head(rows, tse_col, tq_col, ln_g, ln_b, basis2, phase2, Wenc2, benc2,
          Wsrc, Wdst, Wpred, bpred, B, NN):
    Bq, H = rows.shape

    def body(rows_ref, tse_ref, tq_ref, g_ref, b_ref, bas_ref, ph_ref,
             W2_ref, b2_ref, Wsrc_ref, Wdst_ref, Wp_ref, bp_ref,
             pos_ref, neg_ref, loss_ref):
        x = rows_ref[...]
        mu = jnp.mean(x, axis=1, keepdims=True)
        xc = x - mu
        var = jnp.mean(xc * xc, axis=1, keepdims=True)
        xn = xc * lax.rsqrt(var + 1e-5) * g_ref[...] + b_ref[...]
        dt = tq_ref[...] - tse_ref[...]
        te = _fast_cos(dt * bas_ref[...] + ph_ref[...])
        W2 = W2_ref[...]
        h = jnp.maximum(
            jnp.dot(xn, W2[0:H], preferred_element_type=jnp.float32)
            + jnp.dot(te, W2[H:2 * H], preferred_element_type=jnp.float32)
            + b2_ref[...], 0.0)
        Wp = Wp_ref[...]
        emb_u = jnp.dot(h[0:B], Wsrc_ref[...],
                        preferred_element_type=jnp.float32)
        emb_v = jnp.dot(h[B:2 * B], Wdst_ref[...],
                        preferred_element_type=jnp.float32)
        emb_n = jnp.dot(h[2 * B:], Wdst_ref[...],
                        preferred_element_type=jnp.float32)
        su = jnp.dot(emb_u, Wp[0:H], preferred_element_type=jnp.float32)
        sv = jnp.dot(emb_v, Wp[H:2 * H], preferred_element_type=jnp.float32)
        sn = jnp.dot(emb_n, Wp[H:2 * H], preferred_element_type=jnp.float32)
        bp = bp_ref[...]
        pos_l = su + sv + bp
        sur = jnp.concatenate([su] * NN, axis=0) if NN > 1 else su
        neg_l = sur + sn + bp
        pos_ref[...] = pos_l
        neg_ref[...] = neg_l
        lap = jnp.maximum(pos_l, 0.0) + jnp.log(1.0 + jnp.exp(-jnp.abs(pos_l)))
        lan = jnp.maximum(neg_l, 0.0) + jnp.log(1.0 + jnp.exp(-jnp.abs(neg_l)))
        loss = (jnp.sum(lap - pos_l) / B + jnp.sum(lan) / (B * NN))
        loss_ref[...] = loss * jnp.ones((1, 1), jnp.float32)

    return pl.pallas_call(
        body,
        out_shape=(
            jax.ShapeDtypeStruct((B, 1), jnp.float32),
            jax.ShapeDtypeStruct((B * NN, 1), jnp.float32),
            jax.ShapeDtypeStruct((1, 1), jnp.float32),
        ),
    )(rows, tse_col, tq_col, ln_g, ln_b, basis2, phase2, Wenc2, benc2,
      Wsrc, Wdst, Wpred, bpred)


def kernel(nfeat, efeat, timestamps, t, basis1, phase1, Wenc1, benc1,
           Wself1, Wneigh1, bconv1, Wself2, Wneigh2, bconv2,
           ln_g, ln_b, basis2, phase2, Wenc2, benc2, Wsrc, Wdst, Wpred,
           bpred, edge_dst, src_max_eid, node_last_eid, src, dst, neg):
    E, DE = efeat.shape
    N, D = nfeat.shape
    H = Wenc1.shape[1]
    B = src.shape[0]
    NN = neg.shape[0] // B

    seg_col = edge_dst.reshape(E, 1)
    seg_row = edge_dst.reshape(1, E)
    ts_col = timestamps.reshape(E, 1)
    r1 = lambda v: v.reshape(1, -1)

    g0 = _gather_rows(nfeat, edge_dst)
    df = _encode(g0, efeat, ts_col, Wenc1, r1(benc1), r1(basis1), r1(phase1))
    sf = _gather_rows(df, src_max_eid)
    df = _conv(sf, df, seg_col, seg_row, Wself1, Wneigh1, r1(bconv1),
               jnp.float32)
    sf = _gather_rows(df, src_max_eid)
    df = _conv(sf, df, seg_col, seg_row, Wself2, Wneigh2, r1(bconv2),
               jnp.float32)

    uvn = jnp.concatenate([src, dst, neg])
    rows, tse = _pred_gather(node_last_eid, uvn, df, timestamps)
    tq = jnp.concatenate([t, t, jnp.tile(t, NN)]).reshape(-1, 1)
    pos_l, neg_l, loss = _head(
        rows, tse.reshape(-1, 1), tq, r1(ln_g), r1(ln_b), r1(basis2),
        r1(phase2), Wenc2, r1(benc2), Wsrc, Wdst, Wpred, r1(bpred), B, NN)
    return (loss.reshape(()), pos_l[:, 0], neg_l[:, 0])


# speedup vs baseline: 1.0068x; 1.0068x over previous
import functools

import jax
import jax.numpy as jnp
from jax import lax
from jax.experimental import pallas as pl
from jax.experimental.pallas import tpu as pltpu
from jax.experimental.pallas import tpu_sc as plsc

_NW = 32


def _gather_rows(table, idx, tc_tiling=True):
    if table.dtype == jnp.bfloat16:
        V, D = table.shape
        t32 = lax.bitcast_convert_type(
            table.reshape(V, D // 2, 2), jnp.int32)
        out32 = _gather_rows(t32, idx, tc_tiling=False)
        return lax.bitcast_convert_type(out32, jnp.bfloat16).reshape(
            idx.shape[0], D)
    B = idx.shape[0]
    V, D = table.shape
    b_per_w = B // _NW
    CH = 80
    K = 5
    n_iter = b_per_w // (K * CH)
    mesh = plsc.VectorSubcoreMesh(core_axis_name="c", subcore_axis_name="s")

    def body(table_hbm, idx_hbm, out_hbm, idx_all, rows_v, sem):
        wid = lax.axis_index("s") * 2 + lax.axis_index("c")
        base = wid * b_per_w
        pltpu.sync_copy(idx_hbm.at[pl.ds(base, b_per_w)], idx_all)

        def step(c, carry):
            off = c * (K * CH)
            hs = [pltpu.async_copy(
                      table_hbm.at[idx_all.at[pl.ds(off + j * CH, CH)]],
                      rows_v.at[pl.ds(j * CH, CH)], sem)
                  for j in range(K)]
            for h in hs:
                h.wait()
            pltpu.sync_copy(rows_v, out_hbm.at[pl.ds(base + off, K * CH)])
            return carry

        lax.fori_loop(0, n_iter, step, 0)

    gk = pl.kernel(
        body,
        out_type=jax.ShapeDtypeStruct((B, D), table.dtype),
        mesh=mesh,
        scratch_types=[
            pltpu.VMEM((b_per_w,), jnp.int32),
            pltpu.VMEM((K * CH, D), table.dtype),
            pltpu.SemaphoreType.DMA,
        ],
        compiler_params=pltpu.CompilerParams(use_tc_tiling_on_sc=tc_tiling),
    )
    return gk(table, idx)


def _pred_gather(node_last_eid, uvn, feat, ts):
    Bq = uvn.shape[0]
    N = node_last_eid.shape[0]
    E, D = feat.shape
    per = Bq // _NW
    mesh = plsc.VectorSubcoreMesh(core_axis_name="c", subcore_axis_name="s")

    def body(nle_hbm, uvn_hbm, feat_hbm, ts_hbm, rows_out, tse_out,
             uv_v, eidx_v, rows_v, tse_v, sem):
        wid = lax.axis_index("s") * 2 + lax.axis_index("c")
        base = wid * per
        pltpu.sync_copy(uvn_hbm.at[pl.ds(base, per)], uv_v)
        pltpu.async_copy(nle_hbm.at[uv_v], eidx_v, sem).wait()
        pltpu.async_copy(feat_hbm.at[eidx_v], rows_v, sem).wait()
        pltpu.async_copy(ts_hbm.at[eidx_v], tse_v, sem).wait()
        pltpu.sync_copy(rows_v, rows_out.at[pl.ds(base, per)])
        pltpu.sync_copy(tse_v, tse_out.at[pl.ds(base, per)])

    gk = pl.kernel(
        body,
        out_type=(
            jax.ShapeDtypeStruct((Bq, D), feat.dtype),
            jax.ShapeDtypeStruct((Bq,), ts.dtype),
        ),
        mesh=mesh,
        scratch_types=[
            pltpu.VMEM((per,), jnp.int32),
            pltpu.VMEM((per,), jnp.int32),
            pltpu.VMEM((per, D), feat.dtype),
            pltpu.VMEM((per,), ts.dtype),
            pltpu.SemaphoreType.DMA,
        ],
    )
    return gk(node_last_eid, uvn, feat, ts)


_BLK = 2560


def _fast_cos(z):
    nf = jnp.floor(z * 0.3183098861837907 + 0.5)
    r = (z - nf * 3.140625) - nf * 9.676535897932795e-4
    r2 = r * r
    p = 1.0 + r2 * (-0.5 + r2 * (4.1666667908e-2 + r2 * (
        -1.3888889225e-3 + r2 * (2.4801587642e-5 + r2 * -2.7557314297e-7))))
    sign = 1.0 - 2.0 * (nf.astype(jnp.int32) & 1).astype(jnp.float32)
    return sign * p


def _encode(g, ef, ts_col, Wenc1, benc1, basis1, phase1):
    E, D = g.shape
    DE = ef.shape[1]
    H = Wenc1.shape[1]
    nb = E // _BLK

    def body(g_ref, ef_ref, ts_ref, W_ref, b_ref, bas_ref, ph_ref, o_ref):
        te = _fast_cos(ts_ref[...] * bas_ref[...] + ph_ref[...])
        W = W_ref[...]
        g = g_ref[...].astype(jnp.float32)
        ef = ef_ref[...].astype(jnp.float32)
        acc = jnp.dot(g, W[0:D], preferred_element_type=jnp.float32)
        acc = acc + jnp.dot(ef, W[D:D + DE],
                            preferred_element_type=jnp.float32)
        acc = acc + jnp.dot(te, W[D + DE:D + DE + H],
                            preferred_element_type=jnp.float32)
        o_ref[...] = jnp.maximum(acc + b_ref[...], 0.0)

    return pl.pallas_call(
        body,
        grid=(nb,),
        in_specs=[
            pl.BlockSpec((_BLK, D), lambda i: (i, 0)),
            pl.BlockSpec((_BLK, DE), lambda i: (i, 0)),
            pl.BlockSpec((_BLK, 1), lambda i: (i, 0)),
            pl.BlockSpec((D + DE + H, H), lambda i: (0, 0)),
            pl.BlockSpec((1, H), lambda i: (0, 0)),
            pl.BlockSpec((1, H), lambda i: (0, 0)),
            pl.BlockSpec((1, H), lambda i: (0, 0)),
        ],
        out_specs=pl.BlockSpec((_BLK, H), lambda i: (i, 0)),
        out_shape=jax.ShapeDtypeStruct((E, H), jnp.float32),
        compiler_params=pltpu.CompilerParams(
            dimension_semantics=("arbitrary",)),
    )(g, ef, ts_col, Wenc1, benc1, basis1, phase1)


_SCH = 256


def _conv(sf, df, seg_col, seg_row, Ws, Wn, bc, out_dtype):
    E, H = df.shape
    nb = E // _BLK

    def body(sf_ref, df_ref, seg_ref, segr_ref, Ws_ref, Wn_ref, b_ref, o_ref,
             csum, ccnt, cseg):
        @pl.when(pl.program_id(0) == 0)
        def _init():
            csum[...] = jnp.zeros_like(csum)
            ccnt[...] = jnp.zeros_like(ccnt)
            cseg[...] = jnp.full_like(cseg, -1)

        seg = seg_ref[...]
        segr = segr_ref[...]
        y = sf_ref[...].astype(jnp.float32)
        ri = lax.broadcasted_iota(jnp.int32, (_SCH, _SCH), 0)
        ci = lax.broadcasted_iota(jnp.int32, (_SCH, _SCH), 1)
        tri = (ri >= ci).astype(jnp.float32)

        carry_sum = csum[...]
        carry_cnt = ccnt[...]
        carry_seg = cseg[...]
        yh = y.astype(jnp.bfloat16).astype(jnp.float32)
        yl = y - yh
        yhl = jnp.concatenate([yh, yl], axis=1)
        pre = []
        for k in range(_BLK // _SCH):
            sc = seg[k * _SCH:(k + 1) * _SCH]
            sr = segr[:, k * _SCH:(k + 1) * _SCH]
            hk = yhl[k * _SCH:(k + 1) * _SCH]
            M = (sc == sr).astype(jnp.float32) * tri
            i2 = jnp.dot(M, hk, preferred_element_type=jnp.float32)
            inc0 = i2[:, 0:H] + i2[:, H:2 * H]
            cnt0 = jnp.sum(M, axis=1, keepdims=True)
            last_seg = jnp.max(sr, axis=1, keepdims=True)
            eql = (sr == last_seg).astype(jnp.float32)
            t2 = jnp.dot(eql, hk, preferred_element_type=jnp.float32)
            tail0 = t2[:, 0:H] + t2[:, H:2 * H]
            tcnt0 = jnp.sum(eql, axis=1, keepdims=True)
            pre.append((sc, inc0, cnt0, last_seg, tail0, tcnt0))
        aggs = []
        for sc, inc0, cnt0, last_seg, tail0, tcnt0 in pre:
            cont = (sc == carry_seg).astype(jnp.float32)
            inc = inc0 + cont * carry_sum
            pos = cnt0 + cont * carry_cnt
            aggs.append(inc / pos)
            contl = (last_seg == carry_seg).astype(jnp.float32)
            carry_sum = tail0 + contl * carry_sum
            carry_cnt = tcnt0 + contl * carry_cnt
            carry_seg = last_seg
        csum[...] = carry_sum
        ccnt[...] = carry_cnt
        cseg[...] = carry_seg

        agg = jnp.concatenate(aggs, axis=0)
        da = jnp.concatenate([df_ref[...].astype(jnp.float32), agg], axis=1)
        Wsn = jnp.concatenate([Ws_ref[...], Wn_ref[...]], axis=0)
        acc = jnp.dot(da, Wsn, preferred_element_type=jnp.float32)
        o_ref[...] = jnp.maximum(acc + b_ref[...], 0.0).astype(out_dtype)

    return pl.pallas_call(
        body,
        grid=(nb,),
        in_specs=[
            pl.BlockSpec((_BLK, H), lambda i: (i, 0)),
            pl.BlockSpec((_BLK, H), lambda i: (i, 0)),
            pl.BlockSpec((_BLK, 1), lambda i: (i, 0)),
            pl.BlockSpec((1, _BLK), lambda i: (0, i)),
            pl.BlockSpec((H, H), lambda i: (0, 0)),
            pl.BlockSpec((H, H), lambda i: (0, 0)),
            pl.BlockSpec((1, H), lambda i: (0, 0)),
        ],
        out_specs=pl.BlockSpec((_BLK, H), lambda i: (i, 0)),
        out_shape=jax.ShapeDtypeStruct((E, H), out_dtype),
        scratch_shapes=[
            pltpu.VMEM((1, H), jnp.float32),
            pltpu.VMEM((1, 1), jnp.float32),
            pltpu.VMEM((1, 1), jnp.int32),
        ],
        compiler_params=pltpu.CompilerParams(
            dimension_semantics=("arbitrary",)),
    )(sf, df, seg_col, seg_row, Ws, Wn, bc)


def _head(rows, tse_col, tq_col, ln_g, ln_b, basis2, phase2, Wenc2, benc2,
          Wsrc, Wdst, Wpred, bpred, B, NN):
    Bq, H = rows.shape

    def body(rows_ref, tse_ref, tq_ref, g_ref, b_ref, bas_ref, ph_ref,
             W2_ref, b2_ref, Wsrc_ref, Wdst_ref, Wp_ref, bp_ref,
             pos_ref, neg_ref, loss_ref):
        x = rows_ref[...]
        mu = jnp.mean(x, axis=1, keepdims=True)
        xc = x - mu
        var = jnp.mean(xc * xc, axis=1, keepdims=True)
        xn = xc * lax.rsqrt(var + 1e-5) * g_ref[...] + b_ref[...]
        dt = tq_ref[...] - tse_ref[...]
        te = _fast_cos(dt * bas_ref[...] + ph_ref[...])
        W2 = W2_ref[...]
        h = jnp.maximum(
            jnp.dot(xn, W2[0:H], preferred_element_type=jnp.float32)
            + jnp.dot(te, W2[H:2 * H], preferred_element_type=jnp.float32)
            + b2_ref[...], 0.0)
        Wp = Wp_ref[...]
        emb_u = jnp.dot(h[0:B], Wsrc_ref[...],
                        preferred_element_type=jnp.float32)
        emb_v = jnp.dot(h[B:2 * B], Wdst_ref[...],
                        preferred_element_type=jnp.float32)
        emb_n = jnp.dot(h[2 * B:], Wdst_ref[...],
                        preferred_element_type=jnp.float32)
        su = jnp.dot(emb_u, Wp[0:H], preferred_element_type=jnp.float32)
        sv = jnp.dot(emb_v, Wp[H:2 * H], preferred_element_type=jnp.float32)
        sn = jnp.dot(emb_n, Wp[H:2 * H], preferred_element_type=jnp.float32)
        bp = bp_ref[...]
        pos_l = su + sv + bp
        sur = jnp.concatenate([su] * NN, axis=0) if NN > 1 else su
        neg_l = sur + sn + bp
        pos_ref[...] = pos_l
        neg_ref[...] = neg_l
        lap = jnp.maximum(pos_l, 0.0) + jnp.log(1.0 + jnp.exp(-jnp.abs(pos_l)))
        lan = jnp.maximum(neg_l, 0.0) + jnp.log(1.0 + jnp.exp(-jnp.abs(neg_l)))
        loss = (jnp.sum(lap - pos_l) / B + jnp.sum(lan) / (B * NN))
        loss_ref[...] = loss * jnp.ones((1, 1), jnp.float32)

    return pl.pallas_call(
        body,
        out_shape=(
            jax.ShapeDtypeStruct((B, 1), jnp.float32),
            jax.ShapeDtypeStruct((B * NN, 1), jnp.float32),
            jax.ShapeDtypeStruct((1, 1), jnp.float32),
        ),
    )(rows, tse_col, tq_col, ln_g, ln_b, basis2, phase2, Wenc2, benc2,
      Wsrc, Wdst, Wpred, bpred)


def kernel(nfeat, efeat, timestamps, t, basis1, phase1, Wenc1, benc1,
           Wself1, Wneigh1, bconv1, Wself2, Wneigh2, bconv2,
           ln_g, ln_b, basis2, phase2, Wenc2, benc2, Wsrc, Wdst, Wpred,
           bpred, edge_dst, src_max_eid, node_last_eid, src, dst, neg):
    E, DE = efeat.shape
    N, D = nfeat.shape
    H = Wenc1.shape[1]
    B = src.shape[0]
    NN = neg.shape[0] // B

    seg_col = edge_dst.reshape(E, 1)
    seg_row = edge_dst.reshape(1, E)
    ts_col = timestamps.reshape(E, 1)
    r1 = lambda v: v.reshape(1, -1)

    g0 = _gather_rows(nfeat, edge_dst)
    df = _encode(g0, efeat, ts_col, Wenc1, r1(benc1), r1(basis1), r1(phase1))
    sf = _gather_rows(df, src_max_eid)
    df = _conv(sf, df, seg_col, seg_row, Wself1, Wneigh1, r1(bconv1),
               jnp.float32)
    sf = _gather_rows(df, src_max_eid)
    df = _conv(sf, df, seg_col, seg_row, Wself2, Wneigh2, r1(bconv2),
               jnp.float32)

    uvn = jnp.concatenate([src, dst, neg])
    rows, tse = _pred_gather(node_last_eid, uvn, df, timestamps)
    tq = jnp.concatenate([t, t, jnp.tile(t, NN)]).reshape(-1, 1)
    pos_l, neg_l, loss = _head(
        rows, tse.reshape(-1, 1), tq, r1(ln_g), r1(ln_b), r1(basis2),
        r1(phase2), Wenc2, r1(benc2), Wsrc, Wdst, Wpred, r1(bpred), B, NN)
    return (loss.reshape(()), pos_l[:, 0], neg_l[:, 0])


# speedup vs baseline: 1.0448x; 1.0377x over previous
import functools

import jax
import jax.numpy as jnp
from jax import lax
from jax.experimental import pallas as pl
from jax.experimental.pallas import tpu as pltpu
from jax.experimental.pallas import tpu_sc as plsc

_NW = 32


def _gather_rows(table, idx, tc_tiling=True):
    if table.dtype == jnp.bfloat16:
        V, D = table.shape
        t32 = lax.bitcast_convert_type(
            table.reshape(V, D // 2, 2), jnp.int32)
        out32 = _gather_rows(t32, idx, tc_tiling=False)
        return lax.bitcast_convert_type(out32, jnp.bfloat16).reshape(
            idx.shape[0], D)
    B = idx.shape[0]
    V, D = table.shape
    b_per_w = B // _NW
    CH = 80
    K = 5
    n_iter = b_per_w // (K * CH)
    mesh = plsc.VectorSubcoreMesh(core_axis_name="c", subcore_axis_name="s")

    def body(table_hbm, idx_hbm, out_hbm, idx_all, rows_a, rows_b, sem_a,
             sem_b):
        wid = lax.axis_index("s") * 2 + lax.axis_index("c")
        base = wid * b_per_w
        pltpu.sync_copy(idx_hbm.at[pl.ds(base, b_per_w)], idx_all)
        G = K * CH

        def fire(chunk, rows_v, sem):
            off = chunk * G
            for j in range(K):
                pltpu.async_copy(
                    table_hbm.at[idx_all.at[pl.ds(off + j * CH, CH)]],
                    rows_v.at[pl.ds(j * CH, CH)], sem)

        def drain_wb(chunk, rows_v, sem):
            off = chunk * G
            for j in range(K):
                pltpu.make_async_copy(
                    table_hbm.at[idx_all.at[pl.ds(off + j * CH, CH)]],
                    rows_v.at[pl.ds(j * CH, CH)], sem).wait()
            pltpu.sync_copy(rows_v, out_hbm.at[pl.ds(base + off, G)])

        fire(0, rows_a, sem_a)

        def step(c2, carry):
            fire(2 * c2 + 1, rows_b, sem_b)
            drain_wb(2 * c2, rows_a, sem_a)
            fire(2 * c2 + 2, rows_a, sem_a)
            drain_wb(2 * c2 + 1, rows_b, sem_b)
            return carry

        lax.fori_loop(0, (n_iter - 1) // 2, step, 0)
        drain_wb(n_iter - 1, rows_a, sem_a)

    gk = pl.kernel(
        body,
        out_type=jax.ShapeDtypeStruct((B, D), table.dtype),
        mesh=mesh,
        scratch_types=[
            pltpu.VMEM((b_per_w,), jnp.int32),
            pltpu.VMEM((K * CH, D), table.dtype),
            pltpu.VMEM((K * CH, D), table.dtype),
            pltpu.SemaphoreType.DMA,
            pltpu.SemaphoreType.DMA,
        ],
        compiler_params=pltpu.CompilerParams(use_tc_tiling_on_sc=tc_tiling),
    )
    return gk(table, idx)


def _pred_gather(node_last_eid, uvn, feat, ts):
    Bq = uvn.shape[0]
    N = node_last_eid.shape[0]
    E, D = feat.shape
    per = Bq // _NW
    mesh = plsc.VectorSubcoreMesh(core_axis_name="c", subcore_axis_name="s")

    def body(nle_hbm, uvn_hbm, feat_hbm, ts_hbm, rows_out, tse_out,
             uv_v, eidx_v, rows_v, tse_v, sem):
        wid = lax.axis_index("s") * 2 + lax.axis_index("c")
        base = wid * per
        pltpu.sync_copy(uvn_hbm.at[pl.ds(base, per)], uv_v)
        pltpu.async_copy(nle_hbm.at[uv_v], eidx_v, sem).wait()
        pltpu.async_copy(feat_hbm.at[eidx_v], rows_v, sem).wait()
        pltpu.async_copy(ts_hbm.at[eidx_v], tse_v, sem).wait()
        pltpu.sync_copy(rows_v, rows_out.at[pl.ds(base, per)])
        pltpu.sync_copy(tse_v, tse_out.at[pl.ds(base, per)])

    gk = pl.kernel(
        body,
        out_type=(
            jax.ShapeDtypeStruct((Bq, D), feat.dtype),
            jax.ShapeDtypeStruct((Bq,), ts.dtype),
        ),
        mesh=mesh,
        scratch_types=[
            pltpu.VMEM((per,), jnp.int32),
            pltpu.VMEM((per,), jnp.int32),
            pltpu.VMEM((per, D), feat.dtype),
            pltpu.VMEM((per,), ts.dtype),
            pltpu.SemaphoreType.DMA,
        ],
    )
    return gk(node_last_eid, uvn, feat, ts)


_BLK = 2560


def _fast_cos(z):
    nf = jnp.floor(z * 0.3183098861837907 + 0.5)
    r = (z - nf * 3.140625) - nf * 9.676535897932795e-4
    r2 = r * r
    p = 1.0 + r2 * (-0.5 + r2 * (4.1666667908e-2 + r2 * (
        -1.3888889225e-3 + r2 * (2.4801587642e-5 + r2 * -2.7557314297e-7))))
    sign = 1.0 - 2.0 * (nf.astype(jnp.int32) & 1).astype(jnp.float32)
    return sign * p


def _encode(g, ef, ts_col, Wenc1, benc1, basis1, phase1):
    E, D = g.shape
    DE = ef.shape[1]
    H = Wenc1.shape[1]
    nb = E // _BLK

    def body(g_ref, ef_ref, ts_ref, W_ref, b_ref, bas_ref, ph_ref, o_ref):
        te = _fast_cos(ts_ref[...] * bas_ref[...] + ph_ref[...])
        W = W_ref[...]
        g = g_ref[...].astype(jnp.float32)
        ef = ef_ref[...].astype(jnp.float32)
        acc = jnp.dot(g, W[0:D], preferred_element_type=jnp.float32)
        acc = acc + jnp.dot(ef, W[D:D + DE],
                            preferred_element_type=jnp.float32)
        acc = acc + jnp.dot(te, W[D + DE:D + DE + H],
                            preferred_element_type=jnp.float32)
        o_ref[...] = jnp.maximum(acc + b_ref[...], 0.0)

    return pl.pallas_call(
        body,
        grid=(nb,),
        in_specs=[
            pl.BlockSpec((_BLK, D), lambda i: (i, 0)),
            pl.BlockSpec((_BLK, DE), lambda i: (i, 0)),
            pl.BlockSpec((_BLK, 1), lambda i: (i, 0)),
            pl.BlockSpec((D + DE + H, H), lambda i: (0, 0)),
            pl.BlockSpec((1, H), lambda i: (0, 0)),
            pl.BlockSpec((1, H), lambda i: (0, 0)),
            pl.BlockSpec((1, H), lambda i: (0, 0)),
        ],
        out_specs=pl.BlockSpec((_BLK, H), lambda i: (i, 0)),
        out_shape=jax.ShapeDtypeStruct((E, H), jnp.float32),
        compiler_params=pltpu.CompilerParams(
            dimension_semantics=("arbitrary",)),
    )(g, ef, ts_col, Wenc1, benc1, basis1, phase1)


_SCH = 256


def _conv(sf, df, seg_col, seg_row, Ws, Wn, bc, out_dtype):
    E, H = df.shape
    nb = E // _BLK

    def body(sf_ref, df_ref, seg_ref, segr_ref, Ws_ref, Wn_ref, b_ref, o_ref,
             csum, ccnt, cseg):
        @pl.when(pl.program_id(0) == 0)
        def _init():
            csum[...] = jnp.zeros_like(csum)
            ccnt[...] = jnp.zeros_like(ccnt)
            cseg[...] = jnp.full_like(cseg, -1)

        seg = seg_ref[...]
        segr = segr_ref[...]
        y = sf_ref[...].astype(jnp.float32)
        ri = lax.broadcasted_iota(jnp.int32, (_SCH, _SCH), 0)
        ci = lax.broadcasted_iota(jnp.int32, (_SCH, _SCH), 1)
        tri = (ri >= ci).astype(jnp.float32)

        carry_sum = csum[...]
        carry_cnt = ccnt[...]
        carry_seg = cseg[...]
        yh = y.astype(jnp.bfloat16).astype(jnp.float32)
        yl = y - yh
        yhl = jnp.concatenate([yh, yl], axis=1)
        pre = []
        for k in range(_BLK // _SCH):
            sc = seg[k * _SCH:(k + 1) * _SCH]
            sr = segr[:, k * _SCH:(k + 1) * _SCH]
            hk = yhl[k * _SCH:(k + 1) * _SCH]
            M = (sc == sr).astype(jnp.float32) * tri
            i2 = jnp.dot(M, hk, preferred_element_type=jnp.float32)
            inc0 = i2[:, 0:H] + i2[:, H:2 * H]
            cnt0 = jnp.sum(M, axis=1, keepdims=True)
            last_seg = jnp.max(sr, axis=1, keepdims=True)
            eql = (sr == last_seg).astype(jnp.float32)
            t2 = jnp.dot(eql, hk, preferred_element_type=jnp.float32)
            tail0 = t2[:, 0:H] + t2[:, H:2 * H]
            tcnt0 = jnp.sum(eql, axis=1, keepdims=True)
            pre.append((sc, inc0, cnt0, last_seg, tail0, tcnt0))
        aggs = []
        for sc, inc0, cnt0, last_seg, tail0, tcnt0 in pre:
            cont = (sc == carry_seg).astype(jnp.float32)
            inc = inc0 + cont * carry_sum
            pos = cnt0 + cont * carry_cnt
            aggs.append(inc / pos)
            contl = (last_seg == carry_seg).astype(jnp.float32)
            carry_sum = tail0 + contl * carry_sum
            carry_cnt = tcnt0 + contl * carry_cnt
            carry_seg = last_seg
        csum[...] = carry_sum
        ccnt[...] = carry_cnt
        cseg[...] = carry_seg

        agg = jnp.concatenate(aggs, axis=0)
        da = jnp.concatenate([df_ref[...].astype(jnp.float32), agg], axis=1)
        Wsn = jnp.concatenate([Ws_ref[...], Wn_ref[...]], axis=0)
        acc = jnp.dot(da, Wsn, preferred_element_type=jnp.float32)
        o_ref[...] = jnp.maximum(acc + b_ref[...], 0.0).astype(out_dtype)

    return pl.pallas_call(
        body,
        grid=(nb,),
        in_specs=[
            pl.BlockSpec((_BLK, H), lambda i: (i, 0)),
            pl.BlockSpec((_BLK, H), lambda i: (i, 0)),
            pl.BlockSpec((_BLK, 1), lambda i: (i, 0)),
            pl.BlockSpec((1, _BLK), lambda i: (0, i)),
            pl.BlockSpec((H, H), lambda i: (0, 0)),
            pl.BlockSpec((H, H), lambda i: (0, 0)),
            pl.BlockSpec((1, H), lambda i: (0, 0)),
        ],
        out_specs=pl.BlockSpec((_BLK, H), lambda i: (i, 0)),
        out_shape=jax.ShapeDtypeStruct((E, H), out_dtype),
        scratch_shapes=[
            pltpu.VMEM((1, H), jnp.float32),
            pltpu.VMEM((1, 1), jnp.float32),
            pltpu.VMEM((1, 1), jnp.int32),
        ],
        compiler_params=pltpu.CompilerParams(
            dimension_semantics=("arbitrary",)),
    )(sf, df, seg_col, seg_row, Ws, Wn, bc)


def _head(rows, tse_col, tq_col, ln_g, ln_b, basis2, phase2, Wenc2, benc2,
          Wsrc, Wdst, Wpred, bpred, B, NN):
    Bq, H = rows.shape

    def body(rows_ref, tse_ref, tq_ref, g_ref, b_ref, bas_ref, ph_ref,
             W2_ref, b2_ref, Wsrc_ref, Wdst_ref, Wp_ref, bp_ref,
             pos_ref, neg_ref, loss_ref):
        x = rows_ref[...]
        mu = jnp.mean(x, axis=1, keepdims=True)
        xc = x - mu
        var = jnp.mean(xc * xc, axis=1, keepdims=True)
        xn = xc * lax.rsqrt(var + 1e-5) * g_ref[...] + b_ref[...]
        dt = tq_ref[...] - tse_ref[...]
        te = _fast_cos(dt * bas_ref[...] + ph_ref[...])
        W2 = W2_ref[...]
        h = jnp.maximum(
            jnp.dot(xn, W2[0:H], preferred_element_type=jnp.float32)
            + jnp.dot(te, W2[H:2 * H], preferred_element_type=jnp.float32)
            + b2_ref[...], 0.0)
        Wp = Wp_ref[...]
        emb_u = jnp.dot(h[0:B], Wsrc_ref[...],
                        preferred_element_type=jnp.float32)
        emb_v = jnp.dot(h[B:2 * B], Wdst_ref[...],
                        preferred_element_type=jnp.float32)
        emb_n = jnp.dot(h[2 * B:], Wdst_ref[...],
                        preferred_element_type=jnp.float32)
        su = jnp.dot(emb_u, Wp[0:H], preferred_element_type=jnp.float32)
        sv = jnp.dot(emb_v, Wp[H:2 * H], preferred_element_type=jnp.float32)
        sn = jnp.dot(emb_n, Wp[H:2 * H], preferred_element_type=jnp.float32)
        bp = bp_ref[...]
        pos_l = su + sv + bp
        sur = jnp.concatenate([su] * NN, axis=0) if NN > 1 else su
        neg_l = sur + sn + bp
        pos_ref[...] = pos_l
        neg_ref[...] = neg_l
        lap = jnp.maximum(pos_l, 0.0) + jnp.log(1.0 + jnp.exp(-jnp.abs(pos_l)))
        lan = jnp.maximum(neg_l, 0.0) + jnp.log(1.0 + jnp.exp(-jnp.abs(neg_l)))
        loss = (jnp.sum(lap - pos_l) / B + jnp.sum(lan) / (B * NN))
        loss_ref[...] = loss * jnp.ones((1, 1), jnp.float32)

    return pl.pallas_call(
        body,
        out_shape=(
            jax.ShapeDtypeStruct((B, 1), jnp.float32),
            jax.ShapeDtypeStruct((B * NN, 1), jnp.float32),
            jax.ShapeDtypeStruct((1, 1), jnp.float32),
        ),
    )(rows, tse_col, tq_col, ln_g, ln_b, basis2, phase2, Wenc2, benc2,
      Wsrc, Wdst, Wpred, bpred)


def kernel(nfeat, efeat, timestamps, t, basis1, phase1, Wenc1, benc1,
           Wself1, Wneigh1, bconv1, Wself2, Wneigh2, bconv2,
           ln_g, ln_b, basis2, phase2, Wenc2, benc2, Wsrc, Wdst, Wpred,
           bpred, edge_dst, src_max_eid, node_last_eid, src, dst, neg):
    E, DE = efeat.shape
    N, D = nfeat.shape
    H = Wenc1.shape[1]
    B = src.shape[0]
    NN = neg.shape[0] // B

    seg_col = edge_dst.reshape(E, 1)
    seg_row = edge_dst.reshape(1, E)
    ts_col = timestamps.reshape(E, 1)
    r1 = lambda v: v.reshape(1, -1)

    g0 = _gather_rows(nfeat, edge_dst)
    df = _encode(g0, efeat, ts_col, Wenc1, r1(benc1), r1(basis1), r1(phase1))
    sf = _gather_rows(df, src_max_eid)
    df = _conv(sf, df, seg_col, seg_row, Wself1, Wneigh1, r1(bconv1),
               jnp.float32)
    sf = _gather_rows(df, src_max_eid)
    df = _conv(sf, df, seg_col, seg_row, Wself2, Wneigh2, r1(bconv2),
               jnp.float32)

    uvn = jnp.concatenate([src, dst, neg])
    rows, tse = _pred_gather(node_last_eid, uvn, df, timestamps)
    tq = jnp.concatenate([t, t, jnp.tile(t, NN)]).reshape(-1, 1)
    pos_l, neg_l, loss = _head(
        rows, tse.reshape(-1, 1), tq, r1(ln_g), r1(ln_b), r1(basis2),
        r1(phase2), Wenc2, r1(benc2), Wsrc, Wdst, Wpred, r1(bpred), B, NN)
    return (loss.reshape(()), pos_l[:, 0], neg_l[:, 0])


# speedup vs baseline: 1.0457x; 1.0008x over previous
import jax
import jax.numpy as jnp
from jax import lax
from jax.experimental import pallas as pl
from jax.experimental.pallas import tpu as pltpu
from jax.experimental.pallas import tpu_sc as plsc

_NW = 32


def _gather_rows(table, idx, tc_tiling=True):
    if table.dtype == jnp.bfloat16:
        V, D = table.shape
        t32 = lax.bitcast_convert_type(
            table.reshape(V, D // 2, 2), jnp.int32)
        out32 = _gather_rows(t32, idx, tc_tiling=False)
        return lax.bitcast_convert_type(out32, jnp.bfloat16).reshape(
            idx.shape[0], D)
    B = idx.shape[0]
    V, D = table.shape
    b_per_w = B // _NW
    CH = 80
    K = 5
    n_iter = b_per_w // (K * CH)
    mesh = plsc.VectorSubcoreMesh(core_axis_name="c", subcore_axis_name="s")

    def body(table_hbm, idx_hbm, out_hbm, idx_all, rows_a, rows_b, sem_a,
             sem_b):
        wid = lax.axis_index("s") * 2 + lax.axis_index("c")
        base = wid * b_per_w
        pltpu.sync_copy(idx_hbm.at[pl.ds(base, b_per_w)], idx_all)
        G = K * CH

        def fire(chunk, rows_v, sem):
            off = chunk * G
            for j in range(K):
                pltpu.async_copy(
                    table_hbm.at[idx_all.at[pl.ds(off + j * CH, CH)]],
                    rows_v.at[pl.ds(j * CH, CH)], sem)

        def drain_wb(chunk, rows_v, sem):
            off = chunk * G
            for j in range(K):
                pltpu.make_async_copy(
                    table_hbm.at[idx_all.at[pl.ds(off + j * CH, CH)]],
                    rows_v.at[pl.ds(j * CH, CH)], sem).wait()
            pltpu.sync_copy(rows_v, out_hbm.at[pl.ds(base + off, G)])

        fire(0, rows_a, sem_a)

        def step(c2, carry):
            fire(2 * c2 + 1, rows_b, sem_b)
            drain_wb(2 * c2, rows_a, sem_a)
            fire(2 * c2 + 2, rows_a, sem_a)
            drain_wb(2 * c2 + 1, rows_b, sem_b)
            return carry

        lax.fori_loop(0, (n_iter - 1) // 2, step, 0)
        drain_wb(n_iter - 1, rows_a, sem_a)

    gk = pl.kernel(
        body,
        out_type=jax.ShapeDtypeStruct((B, D), table.dtype),
        mesh=mesh,
        scratch_types=[
            pltpu.VMEM((b_per_w,), jnp.int32),
            pltpu.VMEM((K * CH, D), table.dtype),
            pltpu.VMEM((K * CH, D), table.dtype),
            pltpu.SemaphoreType.DMA,
            pltpu.SemaphoreType.DMA,
        ],
        compiler_params=pltpu.CompilerParams(use_tc_tiling_on_sc=tc_tiling),
    )
    return gk(table, idx)


def _pred_gather(node_last_eid, uvn, feat, ts):
    Bq = uvn.shape[0]
    N = node_last_eid.shape[0]
    E, D = feat.shape
    per = Bq // _NW
    mesh = plsc.VectorSubcoreMesh(core_axis_name="c", subcore_axis_name="s")

    def body(nle_hbm, uvn_hbm, feat_hbm, ts_hbm, rows_out, tse_out,
             uv_v, eidx_v, rows_v, tse_v, sem):
        wid = lax.axis_index("s") * 2 + lax.axis_index("c")
        base = wid * per
        pltpu.sync_copy(uvn_hbm.at[pl.ds(base, per)], uv_v)
        pltpu.async_copy(nle_hbm.at[uv_v], eidx_v, sem).wait()
        pltpu.async_copy(feat_hbm.at[eidx_v], rows_v, sem).wait()
        pltpu.async_copy(ts_hbm.at[eidx_v], tse_v, sem).wait()
        pltpu.sync_copy(rows_v, rows_out.at[pl.ds(base, per)])
        pltpu.sync_copy(tse_v, tse_out.at[pl.ds(base, per)])

    gk = pl.kernel(
        body,
        out_type=(
            jax.ShapeDtypeStruct((Bq, D), feat.dtype),
            jax.ShapeDtypeStruct((Bq,), ts.dtype),
        ),
        mesh=mesh,
        scratch_types=[
            pltpu.VMEM((per,), jnp.int32),
            pltpu.VMEM((per,), jnp.int32),
            pltpu.VMEM((per, D), feat.dtype),
            pltpu.VMEM((per,), ts.dtype),
            pltpu.SemaphoreType.DMA,
        ],
    )
    return gk(node_last_eid, uvn, feat, ts)


_BLK = 2560


def _fast_cos(z):
    nf = jnp.floor(z * 0.3183098861837907 + 0.5)
    r = (z - nf * 3.140625) - nf * 9.676535897932795e-4
    r2 = r * r
    p = 1.0 + r2 * (-0.5 + r2 * (4.1666667908e-2 + r2 * (
        -1.3888889225e-3 + r2 * (2.4801587642e-5 + r2 * -2.7557314297e-7))))
    sign = 1.0 - 2.0 * (nf.astype(jnp.int32) & 1).astype(jnp.float32)
    return sign * p


def _encode(g, ef, ts_col, Wenc1, benc1, basis1, phase1):
    E, D = g.shape
    DE = ef.shape[1]
    H = Wenc1.shape[1]
    nb = E // _BLK

    def body(g_ref, ef_ref, ts_ref, W_ref, b_ref, bas_ref, ph_ref, o_ref):
        te = _fast_cos(ts_ref[...] * bas_ref[...] + ph_ref[...])
        W = W_ref[...]
        g = g_ref[...].astype(jnp.float32)
        ef = ef_ref[...].astype(jnp.float32)
        acc = jnp.dot(g, W[0:D], preferred_element_type=jnp.float32)
        acc = acc + jnp.dot(ef, W[D:D + DE],
                            preferred_element_type=jnp.float32)
        acc = acc + jnp.dot(te, W[D + DE:D + DE + H],
                            preferred_element_type=jnp.float32)
        o_ref[...] = jnp.maximum(acc + b_ref[...], 0.0)

    return pl.pallas_call(
        body,
        grid=(nb,),
        in_specs=[
            pl.BlockSpec((_BLK, D), lambda i: (i, 0)),
            pl.BlockSpec((_BLK, DE), lambda i: (i, 0)),
            pl.BlockSpec((_BLK, 1), lambda i: (i, 0)),
            pl.BlockSpec((D + DE + H, H), lambda i: (0, 0)),
            pl.BlockSpec((1, H), lambda i: (0, 0)),
            pl.BlockSpec((1, H), lambda i: (0, 0)),
            pl.BlockSpec((1, H), lambda i: (0, 0)),
        ],
        out_specs=pl.BlockSpec((_BLK, H), lambda i: (i, 0)),
        out_shape=jax.ShapeDtypeStruct((E, H), jnp.float32),
        compiler_params=pltpu.CompilerParams(
            dimension_semantics=("arbitrary",)),
    )(g, ef, ts_col, Wenc1, benc1, basis1, phase1)


_SCH = 256


def _conv(sf, df, seg_col, seg_row, Ws, Wn, bc, out_dtype):
    E, H = df.shape
    nb = E // _BLK

    def body(sf_ref, df_ref, seg_ref, segr_ref, Ws_ref, Wn_ref, b_ref, o_ref,
             csum, ccnt, cseg):
        @pl.when(pl.program_id(0) == 0)
        def _init():
            csum[...] = jnp.zeros_like(csum)
            ccnt[...] = jnp.zeros_like(ccnt)
            cseg[...] = jnp.full_like(cseg, -1)

        seg = seg_ref[...]
        segr = segr_ref[...]
        y = sf_ref[...].astype(jnp.float32)
        ri = lax.broadcasted_iota(jnp.int32, (_SCH, _SCH), 0)
        ci = lax.broadcasted_iota(jnp.int32, (_SCH, _SCH), 1)
        tri = (ri >= ci).astype(jnp.float32)

        carry_sum = csum[...]
        carry_cnt = ccnt[...]
        carry_seg = cseg[...]
        yh = y.astype(jnp.bfloat16).astype(jnp.float32)
        yl = y - yh
        yhl = jnp.concatenate([yh, yl], axis=1)
        pre = []
        for k in range(_BLK // _SCH):
            sc = seg[k * _SCH:(k + 1) * _SCH]
            sr = segr[:, k * _SCH:(k + 1) * _SCH]
            hk = yhl[k * _SCH:(k + 1) * _SCH]
            M = (sc == sr).astype(jnp.float32) * tri
            i2 = jnp.dot(M, hk, preferred_element_type=jnp.float32)
            inc0 = i2[:, 0:H] + i2[:, H:2 * H]
            cnt0 = jnp.sum(M, axis=1, keepdims=True)
            last_seg = jnp.max(sr, axis=1, keepdims=True)
            eql = (sr == last_seg).astype(jnp.float32)
            t2 = jnp.dot(eql, hk, preferred_element_type=jnp.float32)
            tail0 = t2[:, 0:H] + t2[:, H:2 * H]
            tcnt0 = jnp.sum(eql, axis=1, keepdims=True)
            pre.append((sc, inc0, cnt0, last_seg, tail0, tcnt0))
        aggs = []
        for sc, inc0, cnt0, last_seg, tail0, tcnt0 in pre:
            cont = (sc == carry_seg).astype(jnp.float32)
            inc = inc0 + cont * carry_sum
            pos = cnt0 + cont * carry_cnt
            aggs.append(inc / pos)
            contl = (last_seg == carry_seg).astype(jnp.float32)
            carry_sum = tail0 + contl * carry_sum
            carry_cnt = tcnt0 + contl * carry_cnt
            carry_seg = last_seg
        csum[...] = carry_sum
        ccnt[...] = carry_cnt
        cseg[...] = carry_seg

        agg = jnp.concatenate(aggs, axis=0)
        da = jnp.concatenate([df_ref[...].astype(jnp.float32), agg], axis=1)
        Wsn = jnp.concatenate([Ws_ref[...], Wn_ref[...]], axis=0)
        acc = jnp.dot(da, Wsn, preferred_element_type=jnp.float32)
        o_ref[...] = jnp.maximum(acc + b_ref[...], 0.0).astype(out_dtype)

    return pl.pallas_call(
        body,
        grid=(nb,),
        in_specs=[
            pl.BlockSpec((_BLK, H), lambda i: (i, 0)),
            pl.BlockSpec((_BLK, H), lambda i: (i, 0)),
            pl.BlockSpec((_BLK, 1), lambda i: (i, 0)),
            pl.BlockSpec((1, _BLK), lambda i: (0, i)),
            pl.BlockSpec((H, H), lambda i: (0, 0)),
            pl.BlockSpec((H, H), lambda i: (0, 0)),
            pl.BlockSpec((1, H), lambda i: (0, 0)),
        ],
        out_specs=pl.BlockSpec((_BLK, H), lambda i: (i, 0)),
        out_shape=jax.ShapeDtypeStruct((E, H), out_dtype),
        scratch_shapes=[
            pltpu.VMEM((1, H), jnp.float32),
            pltpu.VMEM((1, 1), jnp.float32),
            pltpu.VMEM((1, 1), jnp.int32),
        ],
        compiler_params=pltpu.CompilerParams(
            dimension_semantics=("arbitrary",)),
    )(sf, df, seg_col, seg_row, Ws, Wn, bc)


def _head(rows, tse_col, tq_col, ln_g, ln_b, basis2, phase2, Wenc2, benc2,
          Wsrc, Wdst, Wpred, bpred, B, NN):
    Bq, H = rows.shape

    def body(rows_ref, tse_ref, tq_ref, g_ref, b_ref, bas_ref, ph_ref,
             W2_ref, b2_ref, Wsrc_ref, Wdst_ref, Wp_ref, bp_ref,
             pos_ref, neg_ref, loss_ref):
        x = rows_ref[...]
        mu = jnp.mean(x, axis=1, keepdims=True)
        xc = x - mu
        var = jnp.mean(xc * xc, axis=1, keepdims=True)
        xn = xc * lax.rsqrt(var + 1e-5) * g_ref[...] + b_ref[...]
        dt = tq_ref[...] - tse_ref[...]
        te = _fast_cos(dt * bas_ref[...] + ph_ref[...])
        W2 = W2_ref[...]
        h = jnp.maximum(
            jnp.dot(xn, W2[0:H], preferred_element_type=jnp.float32)
            + jnp.dot(te, W2[H:2 * H], preferred_element_type=jnp.float32)
            + b2_ref[...], 0.0)
        Wp = Wp_ref[...]
        emb_u = jnp.dot(h[0:B], Wsrc_ref[...],
                        preferred_element_type=jnp.float32)
        emb_v = jnp.dot(h[B:2 * B], Wdst_ref[...],
                        preferred_element_type=jnp.float32)
        emb_n = jnp.dot(h[2 * B:], Wdst_ref[...],
                        preferred_element_type=jnp.float32)
        su = jnp.dot(emb_u, Wp[0:H], preferred_element_type=jnp.float32)
        sv = jnp.dot(emb_v, Wp[H:2 * H], preferred_element_type=jnp.float32)
        sn = jnp.dot(emb_n, Wp[H:2 * H], preferred_element_type=jnp.float32)
        bp = bp_ref[...]
        pos_l = su + sv + bp
        sur = jnp.concatenate([su] * NN, axis=0) if NN > 1 else su
        neg_l = sur + sn + bp
        pos_ref[...] = pos_l
        neg_ref[...] = neg_l
        lap = jnp.maximum(pos_l, 0.0) + jnp.log(1.0 + jnp.exp(-jnp.abs(pos_l)))
        lan = jnp.maximum(neg_l, 0.0) + jnp.log(1.0 + jnp.exp(-jnp.abs(neg_l)))
        loss = (jnp.sum(lap - pos_l) / B + jnp.sum(lan) / (B * NN))
        loss_ref[...] = loss * jnp.ones((1, 1), jnp.float32)

    return pl.pallas_call(
        body,
        out_shape=(
            jax.ShapeDtypeStruct((B, 1), jnp.float32),
            jax.ShapeDtypeStruct((B * NN, 1), jnp.float32),
            jax.ShapeDtypeStruct((1, 1), jnp.float32),
        ),
    )(rows, tse_col, tq_col, ln_g, ln_b, basis2, phase2, Wenc2, benc2,
      Wsrc, Wdst, Wpred, bpred)


def kernel(nfeat, efeat, timestamps, t, basis1, phase1, Wenc1, benc1,
           Wself1, Wneigh1, bconv1, Wself2, Wneigh2, bconv2,
           ln_g, ln_b, basis2, phase2, Wenc2, benc2, Wsrc, Wdst, Wpred,
           bpred, edge_dst, src_max_eid, node_last_eid, src, dst, neg):
    E, DE = efeat.shape
    N, D = nfeat.shape
    H = Wenc1.shape[1]
    B = src.shape[0]
    NN = neg.shape[0] // B

    seg_col = edge_dst.reshape(E, 1)
    seg_row = edge_dst.reshape(1, E)
    ts_col = timestamps.reshape(E, 1)
    r1 = lambda v: v.reshape(1, -1)

    g0 = _gather_rows(nfeat, edge_dst)
    df = _encode(g0, efeat, ts_col, Wenc1, r1(benc1), r1(basis1), r1(phase1))
    sf = _gather_rows(df, src_max_eid)
    df = _conv(sf, df, seg_col, seg_row, Wself1, Wneigh1, r1(bconv1),
               jnp.float32)
    sf = _gather_rows(df, src_max_eid)
    df = _conv(sf, df, seg_col, seg_row, Wself2, Wneigh2, r1(bconv2),
               jnp.float32)

    uvn = jnp.concatenate([src, dst, neg])
    rows, tse = _pred_gather(node_last_eid, uvn, df, timestamps)
    tq = jnp.concatenate([t, t, jnp.tile(t, NN)]).reshape(-1, 1)
    pos_l, neg_l, loss = _head(
        rows, tse.reshape(-1, 1), tq, r1(ln_g), r1(ln_b), r1(basis2),
        r1(phase2), Wenc2, r1(benc2), Wsrc, Wdst, Wpred, r1(bpred), B, NN)
    return (loss.reshape(()), pos_l[:, 0], neg_l[:, 0])
